# baseline (speedup 1.0000x reference)
import jax
import jax.numpy as jnp
import numpy as np
from jax.experimental import pallas as pl
from jax.experimental.pallas import tpu as pltpu

T_BLK = 128


def _esq_kernel(e_ref, o_ref):
    for i in range(e_ref.shape[0]):
        o_ref[i, :] = jnp.sum(e_ref[i] ** 2, axis=1)


def _rvq_block_kernel(x_ref, e_ref, ehi_ref, emid_ref, elo_ref, esq_ref,
                      q_ref, i_ref, l_ref):
    t = pl.program_id(1)
    n_q, bins, dim = e_ref.shape

    @pl.when(t == 0)
    def _init():
        l_ref[...] = jnp.zeros_like(l_ref)

    r = x_ref[0].T
    qacc = jnp.zeros(r.shape, jnp.float32)
    for i in range(n_q):
        e = e_ref[i]
        esq = esq_ref[i]
        fsq = jnp.sum(r ** 2, axis=1, keepdims=True)
        cross = jax.lax.dot_general(
            r, e, (((1,), (1,)), ((), ())),
            preferred_element_type=jnp.float32)
        dists = fsq - 2.0 * cross + esq[None, :]
        idx = jnp.argmin(dists, axis=1)
        onehot = (idx[:, None] == jax.lax.broadcasted_iota(
            jnp.int32, (r.shape[0], bins), 1)).astype(jnp.bfloat16)
        def sel(tref):
            return jax.lax.dot_general(
                onehot, tref[i], (((1,), (0,)), ((), ())),
                preferred_element_type=jnp.float32)
        q = (sel(ehi_ref) + sel(emid_ref)) + sel(elo_ref)
        diff = q - r
        l_ref[0, i, :] += jnp.sum(diff * diff, axis=0)
        i_ref[0, i, :] = idx
        r = r - q
        qacc = qacc + q
    q_ref[0] = qacc.T


def kernel(x, frame_rate, embed):
    B, D, T = x.shape
    n_q, bins, dim = embed.shape
    grid = (B, T // T_BLK)
    def _trunc_bf16(v):
        bits = jax.lax.bitcast_convert_type(v, jnp.uint32) & jnp.uint32(0xFFFF0000)
        return jax.lax.bitcast_convert_type(bits, jnp.float32)
    h32 = _trunc_bf16(embed)
    r1 = embed - h32
    m32 = _trunc_bf16(r1)
    ehi = h32.astype(jnp.bfloat16)
    emid = m32.astype(jnp.bfloat16)
    elo = (r1 - m32).astype(jnp.bfloat16)
    esq = pl.pallas_call(
        _esq_kernel,
        out_shape=jax.ShapeDtypeStruct((n_q, bins), jnp.float32),
    )(embed)
    cb_spec = pl.BlockSpec((n_q, bins, dim), lambda b, t: (0, 0, 0))
    q_out, i_out, l_out = pl.pallas_call(
        _rvq_block_kernel,
        grid=grid,
        in_specs=[
            pl.BlockSpec((1, D, T_BLK), lambda b, t: (b, 0, t)),
            cb_spec, cb_spec, cb_spec, cb_spec,
            pl.BlockSpec((n_q, bins), lambda b, t: (0, 0)),
        ],
        out_specs=[
            pl.BlockSpec((1, D, T_BLK), lambda b, t: (b, 0, t)),
            pl.BlockSpec((1, n_q, T_BLK), lambda b, t: (b, 0, t)),
            pl.BlockSpec((1, n_q, dim), lambda b, t: (b, 0, 0)),
        ],
        out_shape=[
            jax.ShapeDtypeStruct((B, D, T), jnp.float32),
            jax.ShapeDtypeStruct((B, n_q, T), jnp.int32),
            jax.ShapeDtypeStruct((B, n_q, dim), jnp.float32),
        ],
        compiler_params=pltpu.CompilerParams(
            dimension_semantics=("parallel", "arbitrary")),
    )(x, embed, ehi, emid, elo, esq)
    indices = jnp.transpose(i_out, (1, 0, 2))
    losses = jnp.sum(l_out, axis=(0, 2)) / (B * T * D)
    bandwidth = jnp.asarray(
        n_q * np.log2(bins) * frame_rate / 1000.0, dtype=jnp.float32)
    return q_out, indices, losses, bandwidth


# speedup vs baseline: 1.4166x; 1.4166x over previous
import jax
import jax.numpy as jnp
import numpy as np
from jax.experimental import pallas as pl
from jax.experimental.pallas import tpu as pltpu

T_BLK = 256


def _esq_kernel(e_ref, o_ref):
    for i in range(e_ref.shape[0]):
        o_ref[i, :] = jnp.sum(e_ref[i] ** 2, axis=1)


def _rvq_block_kernel(x_ref, e_ref, ehi_ref, emid_ref, elo_ref, esq_ref,
                      q_ref, i_ref, l_ref):
    t = pl.program_id(1)
    n_q, bins, dim = e_ref.shape

    @pl.when(t == 0)
    def _init():
        l_ref[...] = jnp.zeros_like(l_ref)

    r = x_ref[0].T
    qacc = jnp.zeros(r.shape, jnp.float32)
    for i in range(n_q):
        e = e_ref[i]
        esq = esq_ref[i]
        fsq = jnp.sum(r ** 2, axis=1, keepdims=True)
        cross2 = jax.lax.dot_general(
            r + r, e, (((1,), (1,)), ((), ())),
            preferred_element_type=jnp.float32)
        dists = fsq - cross2 + esq[None, :]
        m = jnp.min(dists, axis=1, keepdims=True)
        iota = jax.lax.broadcasted_iota(jnp.int32, dists.shape, 1)
        idx = jnp.min(jnp.where(dists == m, iota, bins), axis=1)
        onehot = (idx[:, None] == jax.lax.broadcasted_iota(
            jnp.int32, (r.shape[0], bins), 1)).astype(jnp.bfloat16)
        def sel(tref):
            return jax.lax.dot_general(
                onehot, tref[i], (((1,), (0,)), ((), ())),
                preferred_element_type=jnp.float32)
        q = (sel(ehi_ref) + sel(emid_ref)) + sel(elo_ref)
        diff = q - r
        l_ref[0, i, :] += jnp.sum(diff * diff, axis=0)
        i_ref[0, i, :] = idx
        r = r - q
        qacc = qacc + q
    q_ref[0] = qacc.T


def kernel(x, frame_rate, embed):
    B, D, T = x.shape
    n_q, bins, dim = embed.shape
    grid = (B, T // T_BLK)
    def _trunc_bf16(v):
        bits = jax.lax.bitcast_convert_type(v, jnp.uint32) & jnp.uint32(0xFFFF0000)
        return jax.lax.bitcast_convert_type(bits, jnp.float32)
    h32 = _trunc_bf16(embed)
    r1 = embed - h32
    m32 = _trunc_bf16(r1)
    ehi = h32.astype(jnp.bfloat16)
    emid = m32.astype(jnp.bfloat16)
    elo = (r1 - m32).astype(jnp.bfloat16)
    esq = pl.pallas_call(
        _esq_kernel,
        out_shape=jax.ShapeDtypeStruct((n_q, bins), jnp.float32),
    )(embed)
    cb_spec = pl.BlockSpec((n_q, bins, dim), lambda b, t: (0, 0, 0))
    q_out, i_out, l_out = pl.pallas_call(
        _rvq_block_kernel,
        grid=grid,
        in_specs=[
            pl.BlockSpec((1, D, T_BLK), lambda b, t: (b, 0, t)),
            cb_spec, cb_spec, cb_spec, cb_spec,
            pl.BlockSpec((n_q, bins), lambda b, t: (0, 0)),
        ],
        out_specs=[
            pl.BlockSpec((1, D, T_BLK), lambda b, t: (b, 0, t)),
            pl.BlockSpec((1, n_q, T_BLK), lambda b, t: (b, 0, t)),
            pl.BlockSpec((1, n_q, dim), lambda b, t: (b, 0, 0)),
        ],
        out_shape=[
            jax.ShapeDtypeStruct((B, D, T), jnp.float32),
            jax.ShapeDtypeStruct((B, n_q, T), jnp.int32),
            jax.ShapeDtypeStruct((B, n_q, dim), jnp.float32),
        ],
        compiler_params=pltpu.CompilerParams(
            dimension_semantics=("parallel", "arbitrary")),
    )(x, embed, ehi, emid, elo, esq)
    indices = jnp.transpose(i_out, (1, 0, 2))
    losses = jnp.sum(l_out, axis=(0, 2)) / (B * T * D)
    bandwidth = jnp.asarray(
        n_q * np.log2(bins) * frame_rate / 1000.0, dtype=jnp.float32)
    return q_out, indices, losses, bandwidth
